# Initial kernel scaffold; baseline (speedup 1.0000x reference)
import jax
import jax.numpy as jnp
from jax.experimental import pallas as pl


def kernel(x):
    raise NotImplementedError("write your pallas kernel here")


# speedup vs baseline: 1.4716x; 1.4716x over previous
import jax
import jax.numpy as jnp
from jax.experimental import pallas as pl


def _moddrop_body(x_ref, o_ref):
    xb = x_ref[...]
    sums = jnp.sum(xb, axis=(2, 3))
    gain = jnp.sum((sums != 0).astype(xb.dtype))
    o_ref[...] = xb / gain


@jax.jit
def kernel(x):
    N, C, H, W = x.shape
    return pl.pallas_call(
        _moddrop_body,
        grid=(N,),
        in_specs=[pl.BlockSpec((1, C, H, W), lambda i: (i, 0, 0, 0))],
        out_specs=pl.BlockSpec((1, C, H, W), lambda i: (i, 0, 0, 0)),
        out_shape=jax.ShapeDtypeStruct(x.shape, x.dtype),
    )(x)
